# baseline (speedup 1.0000x reference)
import functools

import jax
import jax.numpy as jnp
from jax import lax
from jax.experimental import pallas as pl
from jax.experimental.pallas import tpu as pltpu
from jax.experimental.pallas import tpu_sc as plsc

_K = 64
_CAP = 2048
_I32_MIN = -(2 ** 31)
_I32_MAX = 2 ** 31 - 1
_NC = 2
_NS = 16


def _to_key(x):
    b = lax.bitcast_convert_type(x, jnp.int32)
    return jnp.where(b >= 0, b, b ^ jnp.int32(0x7FFFFFFF))


def _stage_a_kernel(x_ref, mask_ref, meta_ref, *, k, cap):
    x = x_ref[...]
    rows, d = x.shape
    nw = d // 32
    key = _to_key(x)

    def count_ge(t):
        return jnp.sum((key >= t).astype(jnp.int32), axis=-1, keepdims=True)

    gmax = jnp.max(key.reshape(rows, 32, nw), axis=1)
    hi0 = jnp.max(gmax, axis=-1, keepdims=True)

    def gbody(_, carry):
        lo, hi = carry
        mid = (lo >> 1) + (hi >> 1) + ((lo | hi) & 1)
        cnt = jnp.sum((gmax >= mid).astype(jnp.int32), axis=-1, keepdims=True)
        ge = cnt >= k
        return jnp.where(ge, mid, lo), jnp.where(ge, hi, mid - 1)

    lo0, _ = lax.fori_loop(0, 32, gbody,
                           (jnp.full((rows, 1), _I32_MIN, jnp.int32), hi0))
    c0 = count_ge(lo0)

    def wcond(carry):
        i, lo, hi, cnt = carry
        return jnp.logical_and(i < 40, jnp.any((cnt > cap) & (lo < hi)))

    def wbody(carry):
        i, lo, hi, cnt = carry
        act = (cnt > cap) & (lo < hi)
        mid = (lo >> 1) + (hi >> 1) + ((lo | hi) & 1)
        cm = count_ge(mid)
        ge = cm >= k
        lo_n = jnp.where(act & ge, mid, lo)
        hi_n = jnp.where(act & jnp.logical_not(ge), mid - 1, hi)
        cnt_n = jnp.where(act & ge, cm, cnt)
        return i + 1, lo_n, hi_n, cnt_n

    _, lo, hi, cnt = lax.while_loop(wcond, wbody,
                                    (jnp.int32(0), lo0, hi0, c0))

    m = (key >= lo).reshape(rows, 32, nw)
    acc = jnp.zeros((rows, nw), jnp.int32)
    for j in range(32):
        acc = acc | (m[:, j, :].astype(jnp.int32) << j)
    mask_ref[...] = acc
    meta_ref[...] = jnp.concatenate(
        [lo, hi, cnt, jnp.zeros((rows, 125), jnp.int32)], axis=1)


def _build_stage_b(n_rows, d):
    nw = d // 32
    wpr = n_rows // (_NC * _NS)
    mesh = plsc.VectorSubcoreMesh(core_axis_name="c", subcore_axis_name="s")

    @functools.partial(
        pl.kernel, mesh=mesh,
        out_type=jax.ShapeDtypeStruct((n_rows,), jnp.int32),
        compiler_params=pltpu.CompilerParams(needs_layout_passes=False,
                                             use_tc_tiling_on_sc=True),
        scratch_types=[
            pltpu.VMEM((d,), jnp.float32),
            pltpu.VMEM((nw,), jnp.int32),
            pltpu.VMEM((128,), jnp.int32),
            pltpu.VMEM((nw + 48,), jnp.int32),
            pltpu.VMEM((nw + 48,), jnp.int32),
            pltpu.VMEM((_CAP + 48,), jnp.int32),
            pltpu.VMEM((wpr + 16,), jnp.int32),
        ])
    def stage_b(feat_hbm, mask_hbm, meta_hbm, t_hbm,
                rowv, maskv, metav, nzv, nzi, candk, tbuf):
        wid = lax.axis_index("s") * _NC + lax.axis_index("c")
        base = wid * wpr
        l16 = lax.broadcasted_iota(jnp.int32, (16,), 0)

        def popcount(mask):
            return plsc.all_reduce_population_count(mask)[0]

        def compact_append(ref, keys, vals, mask, off):
            sk = jnp.where(mask, keys, _I32_MIN)
            sv = plsc.sort_key_val(sk, vals, descending=True)
            ref[0][pl.ds(off, 16)] = sv[0]
            if ref[1] is not None:
                ref[1][pl.ds(off, 16)] = sv[1]
            return off + popcount(mask)

        def row_body(r, carry):
            row = base + r
            pltpu.sync_copy(meta_hbm.at[row], metav)
            mv = metav[pl.ds(0, 16)]
            lo = mv[0]
            hi = mv[1]

            def sc_path(lo, hi):
                pltpu.sync_copy(feat_hbm.at[row], rowv)
                pltpu.sync_copy(mask_hbm.at[row], maskv)

                def p1(i, off):
                    wv = maskv[pl.ds(i * 16, 16)]
                    mnz = wv != 0
                    return compact_append((nzi, nzv), i * 16 + l16, wv,
                                          mnz, off)

                nnz = lax.fori_loop(0, nw // 16, p1, jnp.int32(0))

                def p2(bchunk, coff):
                    wv = nzv[pl.ds(bchunk * 16, 16)]
                    wi = nzi[pl.ds(bchunk * 16, 16)]
                    valid = (bchunk * 16 + l16) < nnz

                    def bitstep(j, coff):
                        mbit = (((wv >> j) & 1) == 1) & valid
                        eidx = jnp.where(mbit, j * nw + wi, 0)
                        vals = plsc.load_gather(rowv, [eidx])
                        kk = _to_key(vals)
                        return compact_append((candk, None), kk, l16,
                                               mbit, coff)

                    return lax.fori_loop(0, 32, bitstep, coff)

                ccnt = lax.fori_loop(0, (nnz + 15) >> 4, p2, jnp.int32(0))
                nb = (ccnt + 15) >> 4

                def bisect(_, carry):
                    lo_, hi_ = carry
                    mid = (lo_ >> 1) + (hi_ >> 1) + ((lo_ | hi_) & 1)

                    def cb(m2, acc):
                        kv = candk[pl.ds(m2 * 16, 16)]
                        vm = (m2 * 16 + l16) < ccnt
                        return acc + popcount(vm & (kv >= mid))

                    cm = lax.fori_loop(0, nb, cb, jnp.int32(0))
                    ge = cm >= _K
                    return (jnp.where(ge, mid, lo_),
                            jnp.where(ge, hi_, mid - 1))

                tlo, _ = lax.fori_loop(0, 32, bisect, (lo, hi))
                return tlo

            t = lax.cond(hi > lo, sc_path, lambda a, b: a, lo, hi)
            plsc.store_scatter(tbuf, [jnp.where(l16 == 0, r, wpr + 8)],
                               jnp.full((16,), t, jnp.int32))
            return carry

        lax.fori_loop(0, wpr, row_body, jnp.int32(0))
        pltpu.sync_copy(tbuf.at[pl.ds(0, wpr)], t_hbm.at[pl.ds(base, wpr)])

    return stage_b


def _stage_c_kernel(x_ref, t_ref, o_ref, *, k):
    x = x_ref[...]
    t = t_ref[...]
    rows, d = x.shape
    key = _to_key(x)
    ge = key >= t
    c_ge = jnp.sum(ge.astype(jnp.int32), axis=-1, keepdims=True)
    relu = jnp.maximum(x, 0.0)
    need_fix = jnp.any((c_ge > k) & (t > 0))

    @pl.when(jnp.logical_not(need_fix))
    def _():
        o_ref[...] = jnp.where(ge, relu, 0.0)

    @pl.when(need_fix)
    def _():
        eq = key == t
        c_eq = jnp.sum(eq.astype(jnp.int32), axis=-1, keepdims=True)
        slots = k - (c_ge - c_eq)
        idx = lax.broadcasted_iota(jnp.int32, x.shape, 1)
        lo2 = jnp.zeros((rows, 1), jnp.int32)
        hi2 = jnp.full((rows, 1), d - 1, jnp.int32)

        def body2(_, carry):
            l, h = carry
            m = (l + h) >> 1
            c = jnp.sum((eq & (idx <= m)).astype(jnp.int32), axis=-1,
                        keepdims=True)
            enough = c >= slots
            return jnp.where(enough, l, m + 1), jnp.where(enough, m, h)

        cut, _ = lax.fori_loop(0, 15, body2, (lo2, hi2))
        keep = (key > t) | (eq & (idx <= cut))
        o_ref[...] = jnp.where(keep, relu, 0.0)


def kernel(features):
    B, L, D = features.shape
    n = B * L
    x = features.reshape(n, D)
    rpb = next(r for r in (64, 32, 16, 8, 4, 2, 1) if n % r == 0)
    nw = D // 32

    maskw, meta = pl.pallas_call(
        functools.partial(_stage_a_kernel, k=_K, cap=_CAP),
        grid=(n // rpb,),
        in_specs=[pl.BlockSpec((rpb, D), lambda i: (i, 0))],
        out_specs=[pl.BlockSpec((rpb, nw), lambda i: (i, 0)),
                   pl.BlockSpec((rpb, 128), lambda i: (i, 0))],
        out_shape=[jax.ShapeDtypeStruct((n, nw), jnp.int32),
                   jax.ShapeDtypeStruct((n, 128), jnp.int32)],
    )(x)

    t = _build_stage_b(n, D)(x, maskw, meta)

    out = pl.pallas_call(
        functools.partial(_stage_c_kernel, k=_K),
        grid=(n // rpb,),
        in_specs=[pl.BlockSpec((rpb, D), lambda i: (i, 0)),
                  pl.BlockSpec((rpb, 1), lambda i: (i, 0))],
        out_specs=pl.BlockSpec((rpb, D), lambda i: (i, 0)),
        out_shape=jax.ShapeDtypeStruct((n, D), jnp.float32),
    )(x, t.reshape(n, 1))
    return out.reshape(B, L, D)


# speedup vs baseline: 1.0507x; 1.0507x over previous
import functools

import jax
import jax.numpy as jnp
from jax import lax
from jax.experimental import pallas as pl
from jax.experimental.pallas import tpu as pltpu
from jax.experimental.pallas import tpu_sc as plsc

_K = 64
_CAP = 2048
_I32_MIN = -(2 ** 31)
_I32_MAX = 2 ** 31 - 1
_NC = 2
_NS = 16


def _to_key(x):
    b = lax.bitcast_convert_type(x, jnp.int32)
    return jnp.where(b >= 0, b, b ^ jnp.int32(0x7FFFFFFF))


def _stage_a_kernel(x_ref, mask_ref, meta_ref, *, k, cap):
    x = x_ref[...]
    rows, d = x.shape
    nw = d // 32
    key = _to_key(x)

    def count_ge(t):
        return jnp.sum((key >= t).astype(jnp.int32), axis=-1, keepdims=True)

    gmax = jnp.max(key.reshape(rows, 32, nw), axis=1)
    hi0 = jnp.max(gmax, axis=-1, keepdims=True)

    def gbody(_, carry):
        lo, hi = carry
        mid = (lo >> 1) + (hi >> 1) + ((lo | hi) & 1)
        cnt = jnp.sum((gmax >= mid).astype(jnp.int32), axis=-1, keepdims=True)
        ge = cnt >= k
        return jnp.where(ge, mid, lo), jnp.where(ge, hi, mid - 1)

    lo0, _ = lax.fori_loop(0, 32, gbody,
                           (jnp.full((rows, 1), _I32_MIN, jnp.int32), hi0))
    c0 = count_ge(lo0)

    def wcond(carry):
        i, lo, hi, cnt = carry
        return jnp.logical_and(i < 40, jnp.any((cnt > cap) & (lo < hi)))

    def wbody(carry):
        i, lo, hi, cnt = carry
        act = (cnt > cap) & (lo < hi)
        mid = (lo >> 1) + (hi >> 1) + ((lo | hi) & 1)
        cm = count_ge(mid)
        ge = cm >= k
        lo_n = jnp.where(act & ge, mid, lo)
        hi_n = jnp.where(act & jnp.logical_not(ge), mid - 1, hi)
        cnt_n = jnp.where(act & ge, cm, cnt)
        return i + 1, lo_n, hi_n, cnt_n

    _, lo, hi, cnt = lax.while_loop(wcond, wbody,
                                    (jnp.int32(0), lo0, hi0, c0))

    m = (key >= lo).reshape(rows, 32, nw)
    acc = jnp.zeros((rows, nw), jnp.int32)
    for j in range(32):
        acc = acc | (m[:, j, :].astype(jnp.int32) << j)
    mask_ref[...] = acc
    meta_ref[...] = jnp.concatenate(
        [lo, hi, cnt, jnp.zeros((rows, 125), jnp.int32)], axis=1)


def _build_stage_b(n_rows, d):
    nw = d // 32
    wpr = n_rows // (_NC * _NS)
    mesh = plsc.VectorSubcoreMesh(core_axis_name="c", subcore_axis_name="s")

    @functools.partial(
        pl.kernel, mesh=mesh,
        out_type=jax.ShapeDtypeStruct((n_rows,), jnp.int32),
        compiler_params=pltpu.CompilerParams(needs_layout_passes=False,
                                             use_tc_tiling_on_sc=True),
        scratch_types=[
            pltpu.VMEM((d,), jnp.float32),
            pltpu.VMEM((d,), jnp.float32),
            pltpu.SemaphoreType.DMA,
            pltpu.SemaphoreType.DMA,
            pltpu.VMEM((nw,), jnp.int32),
            pltpu.VMEM((128,), jnp.int32),
            pltpu.VMEM((nw + 48,), jnp.int32),
            pltpu.VMEM((nw + 48,), jnp.int32),
            pltpu.VMEM((_CAP + 48,), jnp.int32),
            pltpu.VMEM((wpr + 16,), jnp.int32),
        ])
    def stage_b(feat_hbm, mask_hbm, meta_hbm, t_hbm,
                rowv0, rowv1, sem0, sem1, maskv, metav, nzv, nzi, candk,
                tbuf):
        wid = lax.axis_index("s") * _NC + lax.axis_index("c")
        base = wid * wpr
        l16 = lax.broadcasted_iota(jnp.int32, (16,), 0)

        def popcount(mask):
            return plsc.all_reduce_population_count(mask)[0]

        def compact_append(ref, keys, vals, mask, off):
            sk = jnp.where(mask, keys, _I32_MIN)
            sv = plsc.sort_key_val(sk, vals, descending=True)
            ref[0][pl.ds(off, 16)] = sv[0]
            if ref[1] is not None:
                ref[1][pl.ds(off, 16)] = sv[1]
            return off + popcount(mask)

        def process(row, rowv):
            pltpu.sync_copy(meta_hbm.at[row], metav)
            mv = metav[pl.ds(0, 16)]
            lo = mv[0]
            hi = mv[1]

            def sc_path(lo, hi):
                pltpu.sync_copy(mask_hbm.at[row], maskv)

                def p1(i, off):
                    wv = maskv[pl.ds(i * 16, 16)]
                    mnz = wv != 0
                    return compact_append((nzi, nzv), i * 16 + l16, wv,
                                          mnz, off)

                nnz = lax.fori_loop(0, nw // 16, p1, jnp.int32(0))

                def p2(bchunk, coff):
                    wv = nzv[pl.ds(bchunk * 16, 16)]
                    wi = nzi[pl.ds(bchunk * 16, 16)]
                    valid = (bchunk * 16 + l16) < nnz

                    def bitstep(j, coff):
                        mbit = (((wv >> j) & 1) == 1) & valid
                        eidx = jnp.where(mbit, j * nw + wi, 0)
                        vals = plsc.load_gather(rowv, [eidx])
                        kk = _to_key(vals)
                        return compact_append((candk, None), kk, l16,
                                               mbit, coff)

                    return lax.fori_loop(0, 32, bitstep, coff)

                ccnt = lax.fori_loop(0, (nnz + 15) >> 4, p2, jnp.int32(0))
                nb = (ccnt + 15) >> 4

                def bisect(_, carry):
                    lo_, hi_ = carry
                    mid = (lo_ >> 1) + (hi_ >> 1) + ((lo_ | hi_) & 1)

                    def cb(m2, acc):
                        kv = candk[pl.ds(m2 * 16, 16)]
                        vm = (m2 * 16 + l16) < ccnt
                        return acc + popcount(vm & (kv >= mid))

                    cm = lax.fori_loop(0, nb, cb, jnp.int32(0))
                    ge = cm >= _K
                    return (jnp.where(ge, mid, lo_),
                            jnp.where(ge, hi_, mid - 1))

                tlo, _ = lax.fori_loop(0, 32, bisect, (lo, hi))
                return tlo

            return lax.cond(hi > lo, sc_path, lambda a, b: a, lo, hi)

        def store_t(r, t):
            plsc.store_scatter(tbuf, [jnp.where(l16 == 0, r, wpr + 8)],
                               jnp.full((16,), t, jnp.int32))

        pltpu.async_copy(feat_hbm.at[base], rowv0, sem0)

        def pair_body(i, carry):
            r0 = 2 * i
            r1 = 2 * i + 1
            pltpu.async_copy(feat_hbm.at[base + r1], rowv1, sem1)
            pltpu.make_async_copy(feat_hbm.at[base + r0], rowv0, sem0).wait()
            store_t(r0, process(base + r0, rowv0))
            nxt = jnp.minimum(base + r0 + 2, n_rows - 1)
            pltpu.async_copy(feat_hbm.at[nxt], rowv0, sem0)
            pltpu.make_async_copy(feat_hbm.at[base + r1], rowv1, sem1).wait()
            store_t(r1, process(base + r1, rowv1))
            return carry

        lax.fori_loop(0, wpr // 2, pair_body, jnp.int32(0))
        pltpu.make_async_copy(feat_hbm.at[base], rowv0, sem0).wait()
        pltpu.sync_copy(tbuf.at[pl.ds(0, wpr)], t_hbm.at[pl.ds(base, wpr)])

    return stage_b


def _stage_c_kernel(x_ref, t_ref, o_ref, *, k):
    x = x_ref[...]
    t = t_ref[...]
    rows, d = x.shape
    key = _to_key(x)
    ge = key >= t
    c_ge = jnp.sum(ge.astype(jnp.int32), axis=-1, keepdims=True)
    relu = jnp.maximum(x, 0.0)
    need_fix = jnp.any((c_ge > k) & (t > 0))

    @pl.when(jnp.logical_not(need_fix))
    def _():
        o_ref[...] = jnp.where(ge, relu, 0.0)

    @pl.when(need_fix)
    def _():
        eq = key == t
        c_eq = jnp.sum(eq.astype(jnp.int32), axis=-1, keepdims=True)
        slots = k - (c_ge - c_eq)
        idx = lax.broadcasted_iota(jnp.int32, x.shape, 1)
        lo2 = jnp.zeros((rows, 1), jnp.int32)
        hi2 = jnp.full((rows, 1), d - 1, jnp.int32)

        def body2(_, carry):
            l, h = carry
            m = (l + h) >> 1
            c = jnp.sum((eq & (idx <= m)).astype(jnp.int32), axis=-1,
                        keepdims=True)
            enough = c >= slots
            return jnp.where(enough, l, m + 1), jnp.where(enough, m, h)

        cut, _ = lax.fori_loop(0, 15, body2, (lo2, hi2))
        keep = (key > t) | (eq & (idx <= cut))
        o_ref[...] = jnp.where(keep, relu, 0.0)


def kernel(features):
    B, L, D = features.shape
    n = B * L
    x = features.reshape(n, D)
    rpb = next(r for r in (64, 32, 16, 8, 4, 2, 1) if n % r == 0)
    nw = D // 32

    maskw, meta = pl.pallas_call(
        functools.partial(_stage_a_kernel, k=_K, cap=_CAP),
        grid=(n // rpb,),
        in_specs=[pl.BlockSpec((rpb, D), lambda i: (i, 0))],
        out_specs=[pl.BlockSpec((rpb, nw), lambda i: (i, 0)),
                   pl.BlockSpec((rpb, 128), lambda i: (i, 0))],
        out_shape=[jax.ShapeDtypeStruct((n, nw), jnp.int32),
                   jax.ShapeDtypeStruct((n, 128), jnp.int32)],
    )(x)

    t = _build_stage_b(n, D)(x, maskw, meta)

    out = pl.pallas_call(
        functools.partial(_stage_c_kernel, k=_K),
        grid=(n // rpb,),
        in_specs=[pl.BlockSpec((rpb, D), lambda i: (i, 0)),
                  pl.BlockSpec((rpb, 1), lambda i: (i, 0))],
        out_specs=pl.BlockSpec((rpb, D), lambda i: (i, 0)),
        out_shape=jax.ShapeDtypeStruct((n, D), jnp.float32),
    )(x, t.reshape(n, 1))
    return out.reshape(B, L, D)
